# baseline (speedup 1.0000x reference)
import functools

import jax
import jax.numpy as jnp
from jax import lax
from jax.experimental import pallas as pl
from jax.experimental.pallas import tpu as pltpu
from jax.experimental.pallas import tpu_sc as plsc

N = 10000
D = 128
E = 320000
G = 64
EPS = 1e-5

NW = 32
EPW = E // NW
CHUNK = 40
NCHUNK = EPW // CHUNK
ROWS_PER_TILE = N // 16
BLK = 2000
NBLK = N // BLK


def _edge_body(k_hbm, q_hbm, v_hbm, src_hbm, dst_hbm, out_hbm,
               acc_sh, dsti, srci, s_v, q_v, v_v, m_v,
               sem_ix, sem_g, sem_s):
    c = lax.axis_index("c")
    s = lax.axis_index("s")
    wid = c * 16 + s

    def _zrow(r, carry):
        for j in range(8):
            m_v[0, r, pl.ds(j * 16, 16)] = jnp.zeros((16,), jnp.float32)
        return carry
    lax.fori_loop(0, CHUNK, _zrow, 0)
    row0 = s * ROWS_PER_TILE
    for t in range(ROWS_PER_TILE // CHUNK):
        pltpu.sync_copy(m_v.at[0],
                        acc_sh.at[pl.ds(row0 + t * CHUNK, CHUNK)])
    _rem = ROWS_PER_TILE % CHUNK
    pltpu.sync_copy(m_v.at[0].at[pl.ds(0, _rem)],
                    acc_sh.at[pl.ds(row0 + ROWS_PER_TILE - _rem, _rem)])
    plsc.subcore_barrier()

    base_w = wid * EPW

    def _issue_idx(ci, buf):
        eb = base_w + ci * CHUNK
        pltpu.async_copy(dst_hbm.at[pl.ds(eb, CHUNK)], dsti.at[buf],
                         sem_ix.at[buf])
        pltpu.async_copy(src_hbm.at[pl.ds(eb, CHUNK)], srci.at[buf],
                         sem_ix.at[buf])

    def _wait_idx(ci, buf):
        pltpu.make_async_copy(dst_hbm.at[pl.ds(base_w, CHUNK)], dsti.at[buf],
                              sem_ix.at[buf]).wait()
        pltpu.make_async_copy(src_hbm.at[pl.ds(base_w, CHUNK)], srci.at[buf],
                              sem_ix.at[buf]).wait()

    def _issue_gathers(buf):
        pltpu.async_copy(k_hbm.at[dsti.at[buf]], s_v.at[buf], sem_g.at[buf])
        pltpu.async_copy(q_hbm.at[srci.at[buf]], q_v.at[buf], sem_g.at[buf])
        pltpu.async_copy(v_hbm.at[srci.at[buf]], v_v.at[buf], sem_g.at[buf])

    def _wait_gathers(buf):
        pltpu.make_async_copy(k_hbm.at[dsti.at[buf]], s_v.at[buf],
                              sem_g.at[buf]).wait()
        pltpu.make_async_copy(q_hbm.at[srci.at[buf]], q_v.at[buf],
                              sem_g.at[buf]).wait()
        pltpu.make_async_copy(v_hbm.at[srci.at[buf]], v_v.at[buf],
                              sem_g.at[buf]).wait()

    def _wait_scatter(buf):
        pltpu.make_async_copy(m_v.at[buf], acc_sh.at[dsti.at[buf]],
                              sem_s.at[buf]).wait()

    def _chunk(i, carry):
        b = jnp.bitwise_and(i, 1)
        nb = 1 - b

        @pl.when(i == 0)
        def _():
            _issue_idx(0, b)
            _wait_idx(0, b)
            _issue_gathers(b)

        @pl.when(i + 1 < NCHUNK)
        def _():
            _issue_idx(i + 1, nb)

        _wait_gathers(b)

        @pl.when(i + 1 < NCHUNK)
        def _():
            _wait_idx(i + 1, nb)
            _issue_gathers(nb)

        @pl.when(i >= 2)
        def _():
            _wait_scatter(b)

        def _row(e, c2):
            for j in range(8):
                sl = pl.ds(j * 16, 16)
                sv = s_v[b, e, sl] + q_v[b, e, sl]
                m_v[b, e, sl] = v_v[b, e, sl] / (1.0 + jnp.exp(-sv))
            return c2
        lax.fori_loop(0, CHUNK, _row, 0)

        pltpu.async_copy(m_v.at[b], acc_sh.at[dsti.at[b]], sem_s.at[b],
                         add=True)
        return carry

    lax.fori_loop(0, NCHUNK, _chunk, 0)
    _wait_scatter(jnp.int32(NCHUNK % 2))
    _wait_scatter(jnp.int32((NCHUNK - 1) % 2))
    plsc.subcore_barrier()

    pltpu.sync_copy(acc_sh.at[pl.ds(s * ROWS_PER_TILE, ROWS_PER_TILE)],
                    out_hbm.at[c, s])


@functools.cache
def _make_edge_sc():
  return pl.kernel(
    _edge_body,
    out_type=jax.ShapeDtypeStruct((2, 16, ROWS_PER_TILE, D), jnp.float32),
    mesh=plsc.VectorSubcoreMesh(core_axis_name="c", subcore_axis_name="s",
                                num_cores=2, num_subcores=16),
    scratch_types=[
        pltpu.VMEM_SHARED((N, D), jnp.float32),
        pltpu.VMEM((2, CHUNK), jnp.int32),
        pltpu.VMEM((2, CHUNK), jnp.int32),
        pltpu.VMEM((2, CHUNK, D), jnp.float32),
        pltpu.VMEM((2, CHUNK, D), jnp.float32),
        pltpu.VMEM((2, CHUNK, D), jnp.float32),
        pltpu.VMEM((2, CHUNK, D), jnp.float32),
        pltpu.SemaphoreType.DMA((2,)),
        pltpu.SemaphoreType.DMA((2,)),
        pltpu.SemaphoreType.DMA((2,)),
    ],
  )


def _edge_sc(k, q, v, src, dst):
    out = _make_edge_sc()(k, q, v, src, dst)
    return out.reshape(2 * N, D)


def _kqvs_first_body(x_ref, w_ref, b_ref, k_ref, q_ref, v_ref, sk_ref):
    y = jnp.dot(x_ref[...], w_ref[...], preferred_element_type=jnp.float32)
    y = y + b_ref[...]
    k_ref[...] = y[:, 0:D]
    q_ref[...] = y[:, D:2 * D]
    v_ref[...] = y[:, 2 * D:3 * D]
    sk_ref[...] = y[:, 3 * D:4 * D]


def _kqvs_bn_body(x_ref, st_ref, g_ref, bb_ref, w_ref, b_ref,
                  k_ref, q_ref, v_ref, sk_ref):
    mean = st_ref[0:1, :] * (1.0 / N)
    var = st_ref[1:2, :] * (1.0 / N) - mean * mean
    h = (x_ref[...] - mean) * lax.rsqrt(var + EPS) * g_ref[...] + bb_ref[...]
    y = jnp.dot(h, w_ref[...], preferred_element_type=jnp.float32)
    y = y + b_ref[...]
    k_ref[...] = y[:, 0:D]
    q_ref[...] = y[:, D:2 * D]
    v_ref[...] = y[:, 2 * D:3 * D]
    sk_ref[...] = y[:, 3 * D:4 * D]


def _res_stats_body(a0_ref, a1_ref, sk_ref, hp_ref, st_ref):
    i = pl.program_id(0)
    hp = a0_ref[...] + a1_ref[...] + sk_ref[...]
    hp_ref[...] = hp
    s1 = jnp.sum(hp, axis=0, keepdims=True)
    s2 = jnp.sum(hp * hp, axis=0, keepdims=True)
    blk = jnp.concatenate([s1, s2, jnp.zeros((6, D), jnp.float32)], axis=0)

    @pl.when(i == 0)
    def _():
        st_ref[...] = blk

    @pl.when(i > 0)
    def _():
        st_ref[...] = st_ref[...] + blk


def _pool_body(hp_ref, st_ref, g_ref, bb_ref, seg_ref,
               gap_ref, gsp_ref, cnt_ref):
    i = pl.program_id(0)
    mean = st_ref[0:1, :] * (1.0 / N)
    var = st_ref[1:2, :] * (1.0 / N) - mean * mean
    h = (hp_ref[...] - mean) * lax.rsqrt(var + EPS) * g_ref[...] + bb_ref[...]
    seg = seg_ref[:, 0:1]
    segT = seg.reshape(1, BLK)
    gid = lax.broadcasted_iota(jnp.int32, (G, BLK), 0)
    mf = (gid == segT).astype(jnp.float32)
    gsum = jnp.dot(mf, h, preferred_element_type=jnp.float32)
    cnt = jnp.broadcast_to(jnp.sum(mf, axis=1, keepdims=True), (G, D))
    rows = [jnp.max(jnp.where(seg == g, h, -jnp.inf), axis=0)
            for g in range(G)]
    gmax = jnp.stack(rows, axis=0)

    @pl.when(i == 0)
    def _():
        gap_ref[...] = gsum
        gsp_ref[...] = gmax
        cnt_ref[...] = cnt

    @pl.when(i > 0)
    def _():
        gap_ref[...] = gap_ref[...] + gsum
        gsp_ref[...] = jnp.maximum(gsp_ref[...], gmax)
        cnt_ref[...] = cnt_ref[...] + cnt


def _bn_rows(x, g, b):
    m = jnp.mean(x, axis=0, keepdims=True)
    v = jnp.mean(x * x, axis=0, keepdims=True) - m * m
    return (x - m) * lax.rsqrt(v + EPS) * g + b


def _mlp_body(gap_ref, gsp_ref, cnt_ref,
              gapg_ref, gapb_ref, gspg_ref, gspb_ref,
              w0_ref, b0_ref, g0_ref, bb0_ref,
              w1_ref, b1_ref, g1_ref, bb1_ref,
              wl_ref, bl_ref, out_ref):
    gap = gap_ref[...] / jnp.maximum(cnt_ref[...], 1.0)
    gap = _bn_rows(gap, gapg_ref[...], gapb_ref[...])
    gsp = _bn_rows(gsp_ref[...], gspg_ref[...], gspb_ref[...])
    out = jnp.concatenate([gap, gsp], axis=1)
    out = jnp.dot(out, w0_ref[...], preferred_element_type=jnp.float32) + b0_ref[...]
    out = (out - jnp.min(out)) / (jnp.max(out) - jnp.min(out))
    out = jnp.maximum(out, 0.0)
    out = _bn_rows(out, g0_ref[...], bb0_ref[...])
    out = jnp.dot(out, w1_ref[...], preferred_element_type=jnp.float32) + b1_ref[...]
    out = (out - jnp.min(out)) / (jnp.max(out) - jnp.min(out))
    out = jnp.maximum(out, 0.0)
    out = _bn_rows(out, g1_ref[...], bb1_ref[...])
    out_ref[...] = (jnp.dot(out, wl_ref[...], preferred_element_type=jnp.float32)
                    + bl_ref[...])


def _row_spec(r, c=D):
    return pl.BlockSpec((r, c), lambda i: (i, 0))


def _full_spec(shape):
    return pl.BlockSpec(shape, lambda i: tuple(0 for _ in shape))


def _kqvs_first(x, wcat, bcat):
    return pl.pallas_call(
        _kqvs_first_body,
        grid=(NBLK,),
        in_specs=[_row_spec(BLK), _full_spec((D, 4 * D)), _full_spec((1, 4 * D))],
        out_specs=[_row_spec(BLK)] * 4,
        out_shape=[jax.ShapeDtypeStruct((N, D), jnp.float32)] * 4,
    )(x, wcat, bcat)


def _kqvs_bn(hp, st, g, b, wcat, bcat):
    return pl.pallas_call(
        _kqvs_bn_body,
        grid=(NBLK,),
        in_specs=[_row_spec(BLK), _full_spec((8, D)), _full_spec((1, D)),
                  _full_spec((1, D)), _full_spec((D, 4 * D)),
                  _full_spec((1, 4 * D))],
        out_specs=[_row_spec(BLK)] * 4,
        out_shape=[jax.ShapeDtypeStruct((N, D), jnp.float32)] * 4,
    )(hp, st, g, b, wcat, bcat)


def _res_stats(agg2, skip):
    return pl.pallas_call(
        _res_stats_body,
        grid=(NBLK,),
        in_specs=[_row_spec(BLK),
                  pl.BlockSpec((BLK, D), lambda i: (i + NBLK, 0)),
                  _row_spec(BLK)],
        out_specs=[_row_spec(BLK), _full_spec((8, D))],
        out_shape=[jax.ShapeDtypeStruct((N, D), jnp.float32),
                   jax.ShapeDtypeStruct((8, D), jnp.float32)],
    )(agg2, agg2, skip)


def _pool(hp, st, g, b, segb):
    return pl.pallas_call(
        _pool_body,
        grid=(NBLK,),
        in_specs=[_row_spec(BLK), _full_spec((8, D)), _full_spec((1, D)),
                  _full_spec((1, D)), _row_spec(BLK)],
        out_specs=[_full_spec((G, D))] * 3,
        out_shape=[jax.ShapeDtypeStruct((G, D), jnp.float32)] * 3,
    )(hp, st, g, b, segb)


def _mlp(gap, gsp, cnt, p):
    w0 = p['lin0_W']
    w1 = p['lin1_W']
    wl = jnp.pad(p['last_W'], ((0, 0), (0, D - 10)))
    bl = jnp.pad(p['last_b'], (0, D - 10)).reshape(1, D)
    args = [gap, gsp, cnt,
            p['gap_g'].reshape(1, D), p['gap_b'].reshape(1, D),
            p['gsp_g'].reshape(1, D), p['gsp_b'].reshape(1, D),
            w0, p['lin0_b'].reshape(1, D),
            p['hbn0_g'].reshape(1, D), p['hbn0_b'].reshape(1, D),
            w1, p['lin1_b'].reshape(1, 64),
            p['hbn1_g'].reshape(1, 64), p['hbn1_b'].reshape(1, 64),
            wl, bl]
    out = pl.pallas_call(
        _mlp_body,
        grid=(1,),
        in_specs=[_full_spec(a.shape) for a in args],
        out_specs=_full_spec((G, D)),
        out_shape=jax.ShapeDtypeStruct((G, D), jnp.float32),
    )(*args)
    return out[:, :10]


def kernel(x, edge_index, batch, params):
    src = edge_index[0]
    dst = edge_index[1]
    segb = jnp.broadcast_to(batch[:, None], (N, D))

    k = q = v = skip = None
    hp = st = None
    for i in range(2):
        wcat = jnp.concatenate([params[f'conv{i}_Wk'], params[f'conv{i}_Wq'],
                                params[f'conv{i}_Wv'], params[f'conv{i}_Wskip']],
                               axis=1)
        bcat = jnp.concatenate([params[f'conv{i}_bk'], params[f'conv{i}_bq'],
                                params[f'conv{i}_bv'], params[f'conv{i}_bias']]
                               ).reshape(1, 4 * D)
        if i == 0:
            k, q, v, skip = _kqvs_first(x, wcat, bcat)
        else:
            k, q, v, skip = _kqvs_bn(hp, st, params[f'cbn{i-1}_g'].reshape(1, D),
                                     params[f'cbn{i-1}_b'].reshape(1, D),
                                     wcat, bcat)
        agg2 = _edge_sc(k, q, v, src, dst)
        hp, st = _res_stats(agg2, skip)

    gap, gsp, cnt = _pool(hp, st, params['cbn1_g'].reshape(1, D),
                          params['cbn1_b'].reshape(1, D), segb)
    return _mlp(gap, gsp, cnt, params)


# speedup vs baseline: 3.3419x; 3.3419x over previous
import functools

import jax
import jax.numpy as jnp
from jax import lax
from jax.experimental import pallas as pl
from jax.experimental.pallas import tpu as pltpu
from jax.experimental.pallas import tpu_sc as plsc

N = 10000
D = 128
E = 320000
G = 64
EPS = 1e-5

NW = 32
EPW = E // NW
CHUNK = 40
NCHUNK = EPW // CHUNK
ROWS_PER_TILE = N // 16
BLK = 2000
NBLK = N // BLK


def _edge_body(k_hbm, q_hbm, v_hbm, src_hbm, dst_hbm, out_hbm,
               acc_sh,
               ixd_a, ixs_a, s_a, q_a, v_a,
               ixd_b, ixs_b, s_b, q_b, v_b,
               m_v, sem_ix_a, sem_g_a, sem_ix_b, sem_g_b):
    c = lax.axis_index("c")
    s = lax.axis_index("s")
    wid = c * 16 + s

    A = (ixd_a, ixs_a, s_a, q_a, v_a, sem_ix_a, sem_g_a)
    B = (ixd_b, ixs_b, s_b, q_b, v_b, sem_ix_b, sem_g_b)

    def _zrow(r, carry):
        for j in range(8):
            m_v[r, pl.ds(j * 16, 16)] = jnp.zeros((16,), jnp.float32)
        return carry
    lax.fori_loop(0, CHUNK, _zrow, 0)
    row0 = s * ROWS_PER_TILE
    for t in range(ROWS_PER_TILE // CHUNK):
        pltpu.sync_copy(m_v, acc_sh.at[pl.ds(row0 + t * CHUNK, CHUNK)])
    _rem = ROWS_PER_TILE % CHUNK
    pltpu.sync_copy(m_v.at[pl.ds(0, _rem)],
                    acc_sh.at[pl.ds(row0 + ROWS_PER_TILE - _rem, _rem)])
    plsc.subcore_barrier()

    base_w = wid * EPW

    def _issue_idx(ci, S):
        ixd, ixs, _, _, _, sem_ix, _ = S
        eb = base_w + ci * CHUNK
        pltpu.async_copy(dst_hbm.at[pl.ds(eb, CHUNK)], ixd, sem_ix)
        pltpu.async_copy(src_hbm.at[pl.ds(eb, CHUNK)], ixs, sem_ix)

    def _wait_idx(S):
        ixd, ixs, _, _, _, sem_ix, _ = S
        pltpu.make_async_copy(dst_hbm.at[pl.ds(base_w, CHUNK)], ixd,
                              sem_ix).wait()
        pltpu.make_async_copy(src_hbm.at[pl.ds(base_w, CHUNK)], ixs,
                              sem_ix).wait()

    def _issue_gathers(S):
        ixd, ixs, s_v, q_v, v_v, _, sem_g = S
        pltpu.async_copy(k_hbm.at[ixd], s_v, sem_g)
        pltpu.async_copy(q_hbm.at[ixs], q_v, sem_g)
        pltpu.async_copy(v_hbm.at[ixs], v_v, sem_g)

    def _wait_gathers(S):
        ixd, ixs, s_v, q_v, v_v, _, sem_g = S
        pltpu.make_async_copy(k_hbm.at[ixd], s_v, sem_g).wait()
        pltpu.make_async_copy(q_hbm.at[ixs], q_v, sem_g).wait()
        pltpu.make_async_copy(v_hbm.at[ixs], v_v, sem_g).wait()

    def _do_chunk(i, S, S_next):
        ixd, ixs, s_v, q_v, v_v, _, _ = S

        @pl.when(i + 1 < NCHUNK)
        def _():
            _issue_idx(i + 1, S_next)

        _wait_gathers(S)

        @pl.when(i + 1 < NCHUNK)
        def _():
            _wait_idx(S_next)
            _issue_gathers(S_next)

        @plsc.parallel_loop(0, CHUNK, 1, unroll=4)
        def _rowp(e):
            for j in range(8):
                sl = pl.ds(j * 16, 16)
                sv = s_v[e, sl] + q_v[e, sl]
                m_v[e, sl] = v_v[e, sl] / (1.0 + jnp.exp(-sv))

        pltpu.sync_copy(m_v, acc_sh.at[ixd], add=True)

    pltpu.sync_copy(dst_hbm.at[pl.ds(base_w, CHUNK)], ixd_a)
    pltpu.sync_copy(src_hbm.at[pl.ds(base_w, CHUNK)], ixs_a)
    _issue_gathers(A)

    def _pair(t, carry):
        _do_chunk(2 * t, A, B)
        _do_chunk(2 * t + 1, B, A)
        return carry

    lax.fori_loop(0, NCHUNK // 2, _pair, 0)
    plsc.subcore_barrier()

    pltpu.sync_copy(acc_sh.at[pl.ds(s * ROWS_PER_TILE, ROWS_PER_TILE)],
                    out_hbm.at[c, s])


@functools.cache
def _make_edge_sc():
  return pl.kernel(
    _edge_body,
    out_type=jax.ShapeDtypeStruct((2, 16, ROWS_PER_TILE, D), jnp.float32),
    mesh=plsc.VectorSubcoreMesh(core_axis_name="c", subcore_axis_name="s",
                                num_cores=2, num_subcores=16),
    scratch_types=[
        pltpu.VMEM_SHARED((N, D), jnp.float32),
        pltpu.VMEM((CHUNK,), jnp.int32),
        pltpu.VMEM((CHUNK,), jnp.int32),
        pltpu.VMEM((CHUNK, D), jnp.float32),
        pltpu.VMEM((CHUNK, D), jnp.float32),
        pltpu.VMEM((CHUNK, D), jnp.float32),
        pltpu.VMEM((CHUNK,), jnp.int32),
        pltpu.VMEM((CHUNK,), jnp.int32),
        pltpu.VMEM((CHUNK, D), jnp.float32),
        pltpu.VMEM((CHUNK, D), jnp.float32),
        pltpu.VMEM((CHUNK, D), jnp.float32),
        pltpu.VMEM((CHUNK, D), jnp.float32),
        pltpu.SemaphoreType.DMA,
        pltpu.SemaphoreType.DMA,
        pltpu.SemaphoreType.DMA,
        pltpu.SemaphoreType.DMA,
    ],
  )


def _edge_sc(k, q, v, src, dst):
    out = _make_edge_sc()(k, q, v, src, dst)
    return out.reshape(2 * N, D)


def _kqvs_first_body(x_ref, w_ref, b_ref, k_ref, q_ref, v_ref, sk_ref):
    y = jnp.dot(x_ref[...], w_ref[...], preferred_element_type=jnp.float32)
    y = y + b_ref[...]
    k_ref[...] = y[:, 0:D]
    q_ref[...] = y[:, D:2 * D]
    v_ref[...] = y[:, 2 * D:3 * D]
    sk_ref[...] = y[:, 3 * D:4 * D]


def _kqvs_bn_body(x_ref, st_ref, g_ref, bb_ref, w_ref, b_ref,
                  k_ref, q_ref, v_ref, sk_ref):
    mean = st_ref[0:1, :] * (1.0 / N)
    var = st_ref[1:2, :] * (1.0 / N) - mean * mean
    h = (x_ref[...] - mean) * lax.rsqrt(var + EPS) * g_ref[...] + bb_ref[...]
    y = jnp.dot(h, w_ref[...], preferred_element_type=jnp.float32)
    y = y + b_ref[...]
    k_ref[...] = y[:, 0:D]
    q_ref[...] = y[:, D:2 * D]
    v_ref[...] = y[:, 2 * D:3 * D]
    sk_ref[...] = y[:, 3 * D:4 * D]


def _res_stats_body(a0_ref, a1_ref, sk_ref, hp_ref, st_ref):
    i = pl.program_id(0)
    hp = a0_ref[...] + a1_ref[...] + sk_ref[...]
    hp_ref[...] = hp
    s1 = jnp.sum(hp, axis=0, keepdims=True)
    s2 = jnp.sum(hp * hp, axis=0, keepdims=True)
    blk = jnp.concatenate([s1, s2, jnp.zeros((6, D), jnp.float32)], axis=0)

    @pl.when(i == 0)
    def _():
        st_ref[...] = blk

    @pl.when(i > 0)
    def _():
        st_ref[...] = st_ref[...] + blk


def _pool_body(hp_ref, st_ref, g_ref, bb_ref, seg_ref,
               gap_ref, gsp_ref, cnt_ref):
    i = pl.program_id(0)
    mean = st_ref[0:1, :] * (1.0 / N)
    var = st_ref[1:2, :] * (1.0 / N) - mean * mean
    h = (hp_ref[...] - mean) * lax.rsqrt(var + EPS) * g_ref[...] + bb_ref[...]
    seg = seg_ref[:, 0:1]
    segT = seg.reshape(1, BLK)
    gid = lax.broadcasted_iota(jnp.int32, (G, BLK), 0)
    mf = (gid == segT).astype(jnp.float32)
    gsum = jnp.dot(mf, h, preferred_element_type=jnp.float32)
    cnt = jnp.broadcast_to(jnp.sum(mf, axis=1, keepdims=True), (G, D))
    rows = [jnp.max(jnp.where(seg == g, h, -jnp.inf), axis=0)
            for g in range(G)]
    gmax = jnp.stack(rows, axis=0)

    @pl.when(i == 0)
    def _():
        gap_ref[...] = gsum
        gsp_ref[...] = gmax
        cnt_ref[...] = cnt

    @pl.when(i > 0)
    def _():
        gap_ref[...] = gap_ref[...] + gsum
        gsp_ref[...] = jnp.maximum(gsp_ref[...], gmax)
        cnt_ref[...] = cnt_ref[...] + cnt


def _bn_rows(x, g, b):
    m = jnp.mean(x, axis=0, keepdims=True)
    v = jnp.mean(x * x, axis=0, keepdims=True) - m * m
    return (x - m) * lax.rsqrt(v + EPS) * g + b


def _mlp_body(gap_ref, gsp_ref, cnt_ref,
              gapg_ref, gapb_ref, gspg_ref, gspb_ref,
              w0_ref, b0_ref, g0_ref, bb0_ref,
              w1_ref, b1_ref, g1_ref, bb1_ref,
              wl_ref, bl_ref, out_ref):
    gap = gap_ref[...] / jnp.maximum(cnt_ref[...], 1.0)
    gap = _bn_rows(gap, gapg_ref[...], gapb_ref[...])
    gsp = _bn_rows(gsp_ref[...], gspg_ref[...], gspb_ref[...])
    out = jnp.concatenate([gap, gsp], axis=1)
    out = jnp.dot(out, w0_ref[...], preferred_element_type=jnp.float32) + b0_ref[...]
    out = (out - jnp.min(out)) / (jnp.max(out) - jnp.min(out))
    out = jnp.maximum(out, 0.0)
    out = _bn_rows(out, g0_ref[...], bb0_ref[...])
    out = jnp.dot(out, w1_ref[...], preferred_element_type=jnp.float32) + b1_ref[...]
    out = (out - jnp.min(out)) / (jnp.max(out) - jnp.min(out))
    out = jnp.maximum(out, 0.0)
    out = _bn_rows(out, g1_ref[...], bb1_ref[...])
    out_ref[...] = (jnp.dot(out, wl_ref[...], preferred_element_type=jnp.float32)
                    + bl_ref[...])


def _row_spec(r, c=D):
    return pl.BlockSpec((r, c), lambda i: (i, 0))


def _full_spec(shape):
    return pl.BlockSpec(shape, lambda i: tuple(0 for _ in shape))


def _kqvs_first(x, wcat, bcat):
    return pl.pallas_call(
        _kqvs_first_body,
        grid=(NBLK,),
        in_specs=[_row_spec(BLK), _full_spec((D, 4 * D)), _full_spec((1, 4 * D))],
        out_specs=[_row_spec(BLK)] * 4,
        out_shape=[jax.ShapeDtypeStruct((N, D), jnp.float32)] * 4,
    )(x, wcat, bcat)


def _kqvs_bn(hp, st, g, b, wcat, bcat):
    return pl.pallas_call(
        _kqvs_bn_body,
        grid=(NBLK,),
        in_specs=[_row_spec(BLK), _full_spec((8, D)), _full_spec((1, D)),
                  _full_spec((1, D)), _full_spec((D, 4 * D)),
                  _full_spec((1, 4 * D))],
        out_specs=[_row_spec(BLK)] * 4,
        out_shape=[jax.ShapeDtypeStruct((N, D), jnp.float32)] * 4,
    )(hp, st, g, b, wcat, bcat)


def _res_stats(agg2, skip):
    return pl.pallas_call(
        _res_stats_body,
        grid=(NBLK,),
        in_specs=[_row_spec(BLK),
                  pl.BlockSpec((BLK, D), lambda i: (i + NBLK, 0)),
                  _row_spec(BLK)],
        out_specs=[_row_spec(BLK), _full_spec((8, D))],
        out_shape=[jax.ShapeDtypeStruct((N, D), jnp.float32),
                   jax.ShapeDtypeStruct((8, D), jnp.float32)],
    )(agg2, agg2, skip)


def _pool(hp, st, g, b, segb):
    return pl.pallas_call(
        _pool_body,
        grid=(NBLK,),
        in_specs=[_row_spec(BLK), _full_spec((8, D)), _full_spec((1, D)),
                  _full_spec((1, D)), _row_spec(BLK)],
        out_specs=[_full_spec((G, D))] * 3,
        out_shape=[jax.ShapeDtypeStruct((G, D), jnp.float32)] * 3,
    )(hp, st, g, b, segb)


def _mlp(gap, gsp, cnt, p):
    w0 = p['lin0_W']
    w1 = p['lin1_W']
    wl = jnp.pad(p['last_W'], ((0, 0), (0, D - 10)))
    bl = jnp.pad(p['last_b'], (0, D - 10)).reshape(1, D)
    args = [gap, gsp, cnt,
            p['gap_g'].reshape(1, D), p['gap_b'].reshape(1, D),
            p['gsp_g'].reshape(1, D), p['gsp_b'].reshape(1, D),
            w0, p['lin0_b'].reshape(1, D),
            p['hbn0_g'].reshape(1, D), p['hbn0_b'].reshape(1, D),
            w1, p['lin1_b'].reshape(1, 64),
            p['hbn1_g'].reshape(1, 64), p['hbn1_b'].reshape(1, 64),
            wl, bl]
    out = pl.pallas_call(
        _mlp_body,
        grid=(1,),
        in_specs=[_full_spec(a.shape) for a in args],
        out_specs=_full_spec((G, D)),
        out_shape=jax.ShapeDtypeStruct((G, D), jnp.float32),
    )(*args)
    return out[:, :10]


def kernel(x, edge_index, batch, params):
    src = edge_index[0]
    dst = edge_index[1]
    segb = jnp.broadcast_to(batch[:, None], (N, D))

    k = q = v = skip = None
    hp = st = None
    for i in range(2):
        wcat = jnp.concatenate([params[f'conv{i}_Wk'], params[f'conv{i}_Wq'],
                                params[f'conv{i}_Wv'], params[f'conv{i}_Wskip']],
                               axis=1)
        bcat = jnp.concatenate([params[f'conv{i}_bk'], params[f'conv{i}_bq'],
                                params[f'conv{i}_bv'], params[f'conv{i}_bias']]
                               ).reshape(1, 4 * D)
        if i == 0:
            k, q, v, skip = _kqvs_first(x, wcat, bcat)
        else:
            k, q, v, skip = _kqvs_bn(hp, st, params[f'cbn{i-1}_g'].reshape(1, D),
                                     params[f'cbn{i-1}_b'].reshape(1, D),
                                     wcat, bcat)
        agg2 = _edge_sc(k, q, v, src, dst)
        hp, st = _res_stats(agg2, skip)

    gap, gsp, cnt = _pool(hp, st, params['cbn1_g'].reshape(1, D),
                          params['cbn1_b'].reshape(1, D), segb)
    return _mlp(gap, gsp, cnt, params)


# speedup vs baseline: 4.0043x; 1.1982x over previous
import functools

import jax
import jax.numpy as jnp
from jax import lax
from jax.experimental import pallas as pl
from jax.experimental.pallas import tpu as pltpu
from jax.experimental.pallas import tpu_sc as plsc

N = 10000
D = 128
E = 320000
G = 64
EPS = 1e-5

NW = 32
EPW = E // NW
CHUNK = 40
NCHUNK = EPW // CHUNK
ROWS_PER_TILE = N // 16
BLK = 2000
NBLK = N // BLK


def _edge_body(k_hbm, q_hbm, v_hbm, src_hbm, dst_hbm, out_hbm,
               acc_sh,
               ixd_a, ixs_a, s_a, q_a, v_a, m_a, ixc_a,
               ixd_b, ixs_b, s_b, q_b, v_b, m_b, ixc_b,
               sem_ix_a, sem_g_a, sem_s_a, sem_ix_b, sem_g_b, sem_s_b,
               sem_ixc_a, sem_ixc_b):
    c = lax.axis_index("c")
    s = lax.axis_index("s")
    wid = c * 16 + s
    m_v = m_a

    A = (ixd_a, ixs_a, s_a, q_a, v_a, sem_ix_a, sem_g_a,
         m_a, ixc_a, sem_s_a, sem_ixc_a)
    B = (ixd_b, ixs_b, s_b, q_b, v_b, sem_ix_b, sem_g_b,
         m_b, ixc_b, sem_s_b, sem_ixc_b)

    def _zrow(r, carry):
        for j in range(8):
            m_v[r, pl.ds(j * 16, 16)] = jnp.zeros((16,), jnp.float32)
        return carry
    lax.fori_loop(0, CHUNK, _zrow, 0)
    row0 = s * ROWS_PER_TILE
    for t in range(ROWS_PER_TILE // CHUNK):
        pltpu.sync_copy(m_v, acc_sh.at[pl.ds(row0 + t * CHUNK, CHUNK)])
    _rem = ROWS_PER_TILE % CHUNK
    pltpu.sync_copy(m_v.at[pl.ds(0, _rem)],
                    acc_sh.at[pl.ds(row0 + ROWS_PER_TILE - _rem, _rem)])
    plsc.subcore_barrier()

    base_w = wid * EPW

    def _issue_idx(ci, S):
        ixd, ixs, _, _, _, sem_ix = S[:6]
        eb = base_w + ci * CHUNK
        pltpu.async_copy(dst_hbm.at[pl.ds(eb, CHUNK)], ixd, sem_ix)
        pltpu.async_copy(src_hbm.at[pl.ds(eb, CHUNK)], ixs, sem_ix)

    def _wait_idx(S):
        ixd, ixs, _, _, _, sem_ix = S[:6]
        pltpu.make_async_copy(dst_hbm.at[pl.ds(base_w, CHUNK)], ixd,
                              sem_ix).wait()
        pltpu.make_async_copy(src_hbm.at[pl.ds(base_w, CHUNK)], ixs,
                              sem_ix).wait()

    def _issue_gathers(S):
        ixd, ixs, s_v, q_v, v_v, _, sem_g = S[:7]
        pltpu.async_copy(k_hbm.at[ixd], s_v, sem_g)
        pltpu.async_copy(q_hbm.at[ixs], q_v, sem_g)
        pltpu.async_copy(v_hbm.at[ixs], v_v, sem_g)

    def _wait_gathers(S):
        ixd, ixs, s_v, q_v, v_v, _, sem_g = S[:7]
        pltpu.make_async_copy(k_hbm.at[ixd], s_v, sem_g).wait()
        pltpu.make_async_copy(q_hbm.at[ixs], q_v, sem_g).wait()
        pltpu.make_async_copy(v_hbm.at[ixs], v_v, sem_g).wait()

    def _wait_scatter(S):
        m, ixc, sem_s = S[7], S[8], S[9]
        pltpu.make_async_copy(m, acc_sh.at[ixc], sem_s).wait()

    def _do_chunk(i, S, S_next):
        ixd, ixs, s_v, q_v, v_v = S[:5]
        m, ixc, sem_s, sem_ixc = S[7], S[8], S[9], S[10]
        eb = base_w + i * CHUNK

        _wait_gathers(S)

        @pl.when(i >= 2)
        def _():
            _wait_scatter(S)

        pltpu.async_copy(dst_hbm.at[pl.ds(eb, CHUNK)], ixc, sem_ixc)

        @pl.when(i + 2 < NCHUNK)
        def _():
            _issue_idx(i + 2, S)

        @pl.when(i + 1 < NCHUNK)
        def _():
            _wait_idx(S_next)
            _issue_gathers(S_next)

        @plsc.parallel_loop(0, CHUNK, 1, unroll=4)
        def _rowp(e):
            for j in range(8):
                sl = pl.ds(j * 16, 16)
                sv = s_v[e, sl] + q_v[e, sl]
                m[e, sl] = v_v[e, sl] / (1.0 + jnp.exp(-sv))

        pltpu.make_async_copy(dst_hbm.at[pl.ds(eb, CHUNK)], ixc,
                              sem_ixc).wait()
        pltpu.async_copy(m, acc_sh.at[ixc], sem_s, add=True)

    pltpu.sync_copy(dst_hbm.at[pl.ds(base_w, CHUNK)], ixd_a)
    pltpu.sync_copy(src_hbm.at[pl.ds(base_w, CHUNK)], ixs_a)
    _issue_gathers(A)
    _issue_idx(1, B)

    def _pair(t, carry):
        _do_chunk(2 * t, A, B)
        _do_chunk(2 * t + 1, B, A)
        return carry

    lax.fori_loop(0, NCHUNK // 2, _pair, 0)
    _wait_scatter(A)
    _wait_scatter(B)
    plsc.subcore_barrier()

    pltpu.sync_copy(acc_sh.at[pl.ds(s * ROWS_PER_TILE, ROWS_PER_TILE)],
                    out_hbm.at[c, s])


@functools.cache
def _make_edge_sc():
  return pl.kernel(
    _edge_body,
    out_type=jax.ShapeDtypeStruct((2, 16, ROWS_PER_TILE, D), jnp.float32),
    mesh=plsc.VectorSubcoreMesh(core_axis_name="c", subcore_axis_name="s",
                                num_cores=2, num_subcores=16),
    scratch_types=[
        pltpu.VMEM_SHARED((N, D), jnp.float32),
        pltpu.VMEM((CHUNK,), jnp.int32),
        pltpu.VMEM((CHUNK,), jnp.int32),
        pltpu.VMEM((CHUNK, D), jnp.float32),
        pltpu.VMEM((CHUNK, D), jnp.float32),
        pltpu.VMEM((CHUNK, D), jnp.float32),
        pltpu.VMEM((CHUNK, D), jnp.float32),
        pltpu.VMEM((CHUNK,), jnp.int32),
        pltpu.VMEM((CHUNK,), jnp.int32),
        pltpu.VMEM((CHUNK,), jnp.int32),
        pltpu.VMEM((CHUNK, D), jnp.float32),
        pltpu.VMEM((CHUNK, D), jnp.float32),
        pltpu.VMEM((CHUNK, D), jnp.float32),
        pltpu.VMEM((CHUNK, D), jnp.float32),
        pltpu.VMEM((CHUNK,), jnp.int32),
        pltpu.SemaphoreType.DMA,
        pltpu.SemaphoreType.DMA,
        pltpu.SemaphoreType.DMA,
        pltpu.SemaphoreType.DMA,
        pltpu.SemaphoreType.DMA,
        pltpu.SemaphoreType.DMA,
        pltpu.SemaphoreType.DMA,
        pltpu.SemaphoreType.DMA,
    ],
  )


def _edge_sc(k, q, v, src, dst):
    out = _make_edge_sc()(k, q, v, src, dst)
    return out.reshape(2 * N, D)


def _kqvs_first_body(x_ref, w_ref, b_ref, k_ref, q_ref, v_ref, sk_ref):
    y = jnp.dot(x_ref[...], w_ref[...], preferred_element_type=jnp.float32)
    y = y + b_ref[...]
    k_ref[...] = y[:, 0:D]
    q_ref[...] = y[:, D:2 * D]
    v_ref[...] = y[:, 2 * D:3 * D]
    sk_ref[...] = y[:, 3 * D:4 * D]


def _kqvs_bn_body(x_ref, st_ref, g_ref, bb_ref, w_ref, b_ref,
                  k_ref, q_ref, v_ref, sk_ref):
    mean = st_ref[0:1, :] * (1.0 / N)
    var = st_ref[1:2, :] * (1.0 / N) - mean * mean
    h = (x_ref[...] - mean) * lax.rsqrt(var + EPS) * g_ref[...] + bb_ref[...]
    y = jnp.dot(h, w_ref[...], preferred_element_type=jnp.float32)
    y = y + b_ref[...]
    k_ref[...] = y[:, 0:D]
    q_ref[...] = y[:, D:2 * D]
    v_ref[...] = y[:, 2 * D:3 * D]
    sk_ref[...] = y[:, 3 * D:4 * D]


def _res_stats_body(a0_ref, a1_ref, sk_ref, hp_ref, st_ref):
    i = pl.program_id(0)
    hp = a0_ref[...] + a1_ref[...] + sk_ref[...]
    hp_ref[...] = hp
    s1 = jnp.sum(hp, axis=0, keepdims=True)
    s2 = jnp.sum(hp * hp, axis=0, keepdims=True)
    blk = jnp.concatenate([s1, s2, jnp.zeros((6, D), jnp.float32)], axis=0)

    @pl.when(i == 0)
    def _():
        st_ref[...] = blk

    @pl.when(i > 0)
    def _():
        st_ref[...] = st_ref[...] + blk


def _pool_body(hp_ref, st_ref, g_ref, bb_ref, seg_ref,
               gap_ref, gsp_ref, cnt_ref):
    i = pl.program_id(0)
    mean = st_ref[0:1, :] * (1.0 / N)
    var = st_ref[1:2, :] * (1.0 / N) - mean * mean
    h = (hp_ref[...] - mean) * lax.rsqrt(var + EPS) * g_ref[...] + bb_ref[...]
    seg = seg_ref[:, 0:1]
    segT = seg.reshape(1, BLK)
    gid = lax.broadcasted_iota(jnp.int32, (G, BLK), 0)
    mf = (gid == segT).astype(jnp.float32)
    gsum = jnp.dot(mf, h, preferred_element_type=jnp.float32)
    cnt = jnp.broadcast_to(jnp.sum(mf, axis=1, keepdims=True), (G, D))
    rows = [jnp.max(jnp.where(seg == g, h, -jnp.inf), axis=0)
            for g in range(G)]
    gmax = jnp.stack(rows, axis=0)

    @pl.when(i == 0)
    def _():
        gap_ref[...] = gsum
        gsp_ref[...] = gmax
        cnt_ref[...] = cnt

    @pl.when(i > 0)
    def _():
        gap_ref[...] = gap_ref[...] + gsum
        gsp_ref[...] = jnp.maximum(gsp_ref[...], gmax)
        cnt_ref[...] = cnt_ref[...] + cnt


def _bn_rows(x, g, b):
    m = jnp.mean(x, axis=0, keepdims=True)
    v = jnp.mean(x * x, axis=0, keepdims=True) - m * m
    return (x - m) * lax.rsqrt(v + EPS) * g + b


def _mlp_body(gap_ref, gsp_ref, cnt_ref,
              gapg_ref, gapb_ref, gspg_ref, gspb_ref,
              w0_ref, b0_ref, g0_ref, bb0_ref,
              w1_ref, b1_ref, g1_ref, bb1_ref,
              wl_ref, bl_ref, out_ref):
    gap = gap_ref[...] / jnp.maximum(cnt_ref[...], 1.0)
    gap = _bn_rows(gap, gapg_ref[...], gapb_ref[...])
    gsp = _bn_rows(gsp_ref[...], gspg_ref[...], gspb_ref[...])
    out = jnp.concatenate([gap, gsp], axis=1)
    out = jnp.dot(out, w0_ref[...], preferred_element_type=jnp.float32) + b0_ref[...]
    out = (out - jnp.min(out)) / (jnp.max(out) - jnp.min(out))
    out = jnp.maximum(out, 0.0)
    out = _bn_rows(out, g0_ref[...], bb0_ref[...])
    out = jnp.dot(out, w1_ref[...], preferred_element_type=jnp.float32) + b1_ref[...]
    out = (out - jnp.min(out)) / (jnp.max(out) - jnp.min(out))
    out = jnp.maximum(out, 0.0)
    out = _bn_rows(out, g1_ref[...], bb1_ref[...])
    out_ref[...] = (jnp.dot(out, wl_ref[...], preferred_element_type=jnp.float32)
                    + bl_ref[...])


def _row_spec(r, c=D):
    return pl.BlockSpec((r, c), lambda i: (i, 0))


def _full_spec(shape):
    return pl.BlockSpec(shape, lambda i: tuple(0 for _ in shape))


def _kqvs_first(x, wcat, bcat):
    return pl.pallas_call(
        _kqvs_first_body,
        grid=(NBLK,),
        in_specs=[_row_spec(BLK), _full_spec((D, 4 * D)), _full_spec((1, 4 * D))],
        out_specs=[_row_spec(BLK)] * 4,
        out_shape=[jax.ShapeDtypeStruct((N, D), jnp.float32)] * 4,
    )(x, wcat, bcat)


def _kqvs_bn(hp, st, g, b, wcat, bcat):
    return pl.pallas_call(
        _kqvs_bn_body,
        grid=(NBLK,),
        in_specs=[_row_spec(BLK), _full_spec((8, D)), _full_spec((1, D)),
                  _full_spec((1, D)), _full_spec((D, 4 * D)),
                  _full_spec((1, 4 * D))],
        out_specs=[_row_spec(BLK)] * 4,
        out_shape=[jax.ShapeDtypeStruct((N, D), jnp.float32)] * 4,
    )(hp, st, g, b, wcat, bcat)


def _res_stats(agg2, skip):
    return pl.pallas_call(
        _res_stats_body,
        grid=(NBLK,),
        in_specs=[_row_spec(BLK),
                  pl.BlockSpec((BLK, D), lambda i: (i + NBLK, 0)),
                  _row_spec(BLK)],
        out_specs=[_row_spec(BLK), _full_spec((8, D))],
        out_shape=[jax.ShapeDtypeStruct((N, D), jnp.float32),
                   jax.ShapeDtypeStruct((8, D), jnp.float32)],
    )(agg2, agg2, skip)


def _pool(hp, st, g, b, segb):
    return pl.pallas_call(
        _pool_body,
        grid=(NBLK,),
        in_specs=[_row_spec(BLK), _full_spec((8, D)), _full_spec((1, D)),
                  _full_spec((1, D)), _row_spec(BLK)],
        out_specs=[_full_spec((G, D))] * 3,
        out_shape=[jax.ShapeDtypeStruct((G, D), jnp.float32)] * 3,
    )(hp, st, g, b, segb)


def _mlp(gap, gsp, cnt, p):
    w0 = p['lin0_W']
    w1 = p['lin1_W']
    wl = jnp.pad(p['last_W'], ((0, 0), (0, D - 10)))
    bl = jnp.pad(p['last_b'], (0, D - 10)).reshape(1, D)
    args = [gap, gsp, cnt,
            p['gap_g'].reshape(1, D), p['gap_b'].reshape(1, D),
            p['gsp_g'].reshape(1, D), p['gsp_b'].reshape(1, D),
            w0, p['lin0_b'].reshape(1, D),
            p['hbn0_g'].reshape(1, D), p['hbn0_b'].reshape(1, D),
            w1, p['lin1_b'].reshape(1, 64),
            p['hbn1_g'].reshape(1, 64), p['hbn1_b'].reshape(1, 64),
            wl, bl]
    out = pl.pallas_call(
        _mlp_body,
        grid=(1,),
        in_specs=[_full_spec(a.shape) for a in args],
        out_specs=_full_spec((G, D)),
        out_shape=jax.ShapeDtypeStruct((G, D), jnp.float32),
    )(*args)
    return out[:, :10]


def kernel(x, edge_index, batch, params):
    src = edge_index[0]
    dst = edge_index[1]
    segb = jnp.broadcast_to(batch[:, None], (N, D))

    k = q = v = skip = None
    hp = st = None
    for i in range(2):
        wcat = jnp.concatenate([params[f'conv{i}_Wk'], params[f'conv{i}_Wq'],
                                params[f'conv{i}_Wv'], params[f'conv{i}_Wskip']],
                               axis=1)
        bcat = jnp.concatenate([params[f'conv{i}_bk'], params[f'conv{i}_bq'],
                                params[f'conv{i}_bv'], params[f'conv{i}_bias']]
                               ).reshape(1, 4 * D)
        if i == 0:
            k, q, v, skip = _kqvs_first(x, wcat, bcat)
        else:
            k, q, v, skip = _kqvs_bn(hp, st, params[f'cbn{i-1}_g'].reshape(1, D),
                                     params[f'cbn{i-1}_b'].reshape(1, D),
                                     wcat, bcat)
        agg2 = _edge_sc(k, q, v, src, dst)
        hp, st = _res_stats(agg2, skip)

    gap, gsp, cnt = _pool(hp, st, params['cbn1_g'].reshape(1, D),
                          params['cbn1_b'].reshape(1, D), segb)
    return _mlp(gap, gsp, cnt, params)


# speedup vs baseline: 4.0406x; 1.0091x over previous
import functools

import jax
import jax.numpy as jnp
from jax import lax
from jax.experimental import pallas as pl
from jax.experimental.pallas import tpu as pltpu
from jax.experimental.pallas import tpu_sc as plsc

N = 10000
D = 128
E = 320000
G = 64
EPS = 1e-5

NW = 32
EPW = E // NW
CHUNK = 40
NCHUNK = EPW // CHUNK
ROWS_PER_TILE = N // 16
BLK = 2000
NBLK = N // BLK


def _edge_body(k_hbm, qv_hbm, src_hbm, dst_hbm, out_hbm,
               acc_sh,
               ixd_a, ixs_a, kb_a, qv_a, m_a, ixc_a,
               ixd_b, ixs_b, kb_b, qv_b, m_b, ixc_b,
               sem_ix_a, sem_g_a, sem_s_a, sem_ix_b, sem_g_b, sem_s_b,
               sem_ixc_a, sem_ixc_b):
    c = lax.axis_index("c")
    s = lax.axis_index("s")
    wid = c * 16 + s
    m_v = m_a

    A = (ixd_a, ixs_a, kb_a, qv_a, sem_ix_a, sem_g_a,
         m_a, ixc_a, sem_s_a, sem_ixc_a)
    B = (ixd_b, ixs_b, kb_b, qv_b, sem_ix_b, sem_g_b,
         m_b, ixc_b, sem_s_b, sem_ixc_b)

    def _zrow(r, carry):
        for j in range(8):
            m_v[r, pl.ds(j * 16, 16)] = jnp.zeros((16,), jnp.float32)
        return carry
    lax.fori_loop(0, CHUNK, _zrow, 0)
    row0 = s * ROWS_PER_TILE
    for t in range(ROWS_PER_TILE // CHUNK):
        pltpu.sync_copy(m_v, acc_sh.at[pl.ds(row0 + t * CHUNK, CHUNK)])
    _rem = ROWS_PER_TILE % CHUNK
    pltpu.sync_copy(m_v.at[pl.ds(0, _rem)],
                    acc_sh.at[pl.ds(row0 + ROWS_PER_TILE - _rem, _rem)])
    plsc.subcore_barrier()

    base_w = wid * EPW

    def _issue_idx(ci, S):
        ixd, ixs, sem_ix = S[0], S[1], S[4]
        eb = base_w + ci * CHUNK
        pltpu.async_copy(dst_hbm.at[pl.ds(eb, CHUNK)], ixd, sem_ix)
        pltpu.async_copy(src_hbm.at[pl.ds(eb, CHUNK)], ixs, sem_ix)

    def _wait_idx(S):
        ixd, ixs, sem_ix = S[0], S[1], S[4]
        pltpu.make_async_copy(dst_hbm.at[pl.ds(base_w, CHUNK)], ixd,
                              sem_ix).wait()
        pltpu.make_async_copy(src_hbm.at[pl.ds(base_w, CHUNK)], ixs,
                              sem_ix).wait()

    def _issue_gathers(S):
        ixd, ixs, kb_v, qv_v, sem_g = S[0], S[1], S[2], S[3], S[5]
        pltpu.async_copy(k_hbm.at[ixd], kb_v, sem_g)
        pltpu.async_copy(qv_hbm.at[ixs], qv_v, sem_g)

    def _wait_gathers(S):
        ixd, ixs, kb_v, qv_v, sem_g = S[0], S[1], S[2], S[3], S[5]
        pltpu.make_async_copy(k_hbm.at[ixd], kb_v, sem_g).wait()
        pltpu.make_async_copy(qv_hbm.at[ixs], qv_v, sem_g).wait()

    def _wait_scatter(S):
        m, ixc, sem_s = S[6], S[7], S[8]
        pltpu.make_async_copy(m, acc_sh.at[ixc], sem_s).wait()

    def _do_chunk(i, S, S_next):
        kb_v, qv_v = S[2], S[3]
        m, ixc, sem_s, sem_ixc = S[6], S[7], S[8], S[9]
        eb = base_w + i * CHUNK

        _wait_gathers(S)

        @pl.when(i >= 2)
        def _():
            _wait_scatter(S)

        pltpu.async_copy(dst_hbm.at[pl.ds(eb, CHUNK)], ixc, sem_ixc)

        @pl.when(i + 2 < NCHUNK)
        def _():
            _issue_idx(i + 2, S)

        @pl.when(i + 1 < NCHUNK)
        def _():
            _wait_idx(S_next)
            _issue_gathers(S_next)

        hi_mask = jnp.full((16,), -65536, jnp.int32)

        def _bf2f(w):
            lo = lax.bitcast_convert_type(jnp.left_shift(w, 16), jnp.float32)
            hi = lax.bitcast_convert_type(jnp.bitwise_and(w, hi_mask),
                                          jnp.float32)
            return lo, hi

        @plsc.parallel_loop(0, CHUNK, 1, unroll=4)
        def _rowp(e):
            for g in range(4):
                q0, q1 = _bf2f(qv_v[e, pl.ds(g * 16, 16)])
                v0, v1 = _bf2f(qv_v[e, pl.ds(64 + g * 16, 16)])
                s0 = kb_v[e, pl.ds(g * 32, 16)] + q0
                s1 = kb_v[e, pl.ds(g * 32 + 16, 16)] + q1
                m[e, pl.ds(g * 32, 16)] = v0 / (1.0 + jnp.exp(-s0))
                m[e, pl.ds(g * 32 + 16, 16)] = v1 / (1.0 + jnp.exp(-s1))

        pltpu.make_async_copy(dst_hbm.at[pl.ds(eb, CHUNK)], ixc,
                              sem_ixc).wait()
        pltpu.async_copy(m, acc_sh.at[ixc], sem_s, add=True)

    pltpu.sync_copy(dst_hbm.at[pl.ds(base_w, CHUNK)], ixd_a)
    pltpu.sync_copy(src_hbm.at[pl.ds(base_w, CHUNK)], ixs_a)
    _issue_gathers(A)
    _issue_idx(1, B)

    def _pair(t, carry):
        _do_chunk(2 * t, A, B)
        _do_chunk(2 * t + 1, B, A)
        return carry

    lax.fori_loop(0, NCHUNK // 2, _pair, 0)
    _wait_scatter(A)
    _wait_scatter(B)
    plsc.subcore_barrier()

    pltpu.sync_copy(acc_sh.at[pl.ds(s * ROWS_PER_TILE, ROWS_PER_TILE)],
                    out_hbm.at[c, s])


@functools.cache
def _make_edge_sc():
  return pl.kernel(
    _edge_body,
    out_type=jax.ShapeDtypeStruct((2, 16, ROWS_PER_TILE, D), jnp.float32),
    mesh=plsc.VectorSubcoreMesh(core_axis_name="c", subcore_axis_name="s",
                                num_cores=2, num_subcores=16),
    scratch_types=[
        pltpu.VMEM_SHARED((N, D), jnp.float32),
        pltpu.VMEM((CHUNK,), jnp.int32),
        pltpu.VMEM((CHUNK,), jnp.int32),
        pltpu.VMEM((CHUNK, D), jnp.float32),
        pltpu.VMEM((CHUNK, D), jnp.int32),
        pltpu.VMEM((CHUNK, D), jnp.float32),
        pltpu.VMEM((CHUNK,), jnp.int32),
        pltpu.VMEM((CHUNK,), jnp.int32),
        pltpu.VMEM((CHUNK,), jnp.int32),
        pltpu.VMEM((CHUNK, D), jnp.float32),
        pltpu.VMEM((CHUNK, D), jnp.int32),
        pltpu.VMEM((CHUNK, D), jnp.float32),
        pltpu.VMEM((CHUNK,), jnp.int32),
        pltpu.SemaphoreType.DMA,
        pltpu.SemaphoreType.DMA,
        pltpu.SemaphoreType.DMA,
        pltpu.SemaphoreType.DMA,
        pltpu.SemaphoreType.DMA,
        pltpu.SemaphoreType.DMA,
        pltpu.SemaphoreType.DMA,
        pltpu.SemaphoreType.DMA,
    ],
  )


def _edge_sc(k, q, v, src, dst):
    qv = jax.lax.bitcast_convert_type(
        jnp.concatenate([q, v], axis=1).astype(jnp.bfloat16).reshape(N, D, 2),
        jnp.int32)
    out = _make_edge_sc()(k, qv, src, dst)
    return out.reshape(2 * N, D)


def _kqvs_first_body(x_ref, w_ref, b_ref, k_ref, q_ref, v_ref, sk_ref):
    y = jnp.dot(x_ref[...], w_ref[...], preferred_element_type=jnp.float32)
    y = y + b_ref[...]
    k_ref[...] = y[:, 0:D]
    q_ref[...] = y[:, D:2 * D]
    v_ref[...] = y[:, 2 * D:3 * D]
    sk_ref[...] = y[:, 3 * D:4 * D]


def _kqvs_bn_body(x_ref, st_ref, g_ref, bb_ref, w_ref, b_ref,
                  k_ref, q_ref, v_ref, sk_ref):
    mean = st_ref[0:1, :] * (1.0 / N)
    var = st_ref[1:2, :] * (1.0 / N) - mean * mean
    h = (x_ref[...] - mean) * lax.rsqrt(var + EPS) * g_ref[...] + bb_ref[...]
    y = jnp.dot(h, w_ref[...], preferred_element_type=jnp.float32)
    y = y + b_ref[...]
    k_ref[...] = y[:, 0:D]
    q_ref[...] = y[:, D:2 * D]
    v_ref[...] = y[:, 2 * D:3 * D]
    sk_ref[...] = y[:, 3 * D:4 * D]


def _res_stats_body(a0_ref, a1_ref, sk_ref, hp_ref, st_ref):
    i = pl.program_id(0)
    hp = a0_ref[...] + a1_ref[...] + sk_ref[...]
    hp_ref[...] = hp
    s1 = jnp.sum(hp, axis=0, keepdims=True)
    s2 = jnp.sum(hp * hp, axis=0, keepdims=True)
    blk = jnp.concatenate([s1, s2, jnp.zeros((6, D), jnp.float32)], axis=0)

    @pl.when(i == 0)
    def _():
        st_ref[...] = blk

    @pl.when(i > 0)
    def _():
        st_ref[...] = st_ref[...] + blk


def _pool_body(hp_ref, st_ref, g_ref, bb_ref, seg_ref,
               gap_ref, gsp_ref, cnt_ref):
    i = pl.program_id(0)
    mean = st_ref[0:1, :] * (1.0 / N)
    var = st_ref[1:2, :] * (1.0 / N) - mean * mean
    h = (hp_ref[...] - mean) * lax.rsqrt(var + EPS) * g_ref[...] + bb_ref[...]
    seg = seg_ref[:, 0:1]
    segT = seg.reshape(1, BLK)
    gid = lax.broadcasted_iota(jnp.int32, (G, BLK), 0)
    mf = (gid == segT).astype(jnp.float32)
    gsum = jnp.dot(mf, h, preferred_element_type=jnp.float32)
    cnt = jnp.broadcast_to(jnp.sum(mf, axis=1, keepdims=True), (G, D))
    rows = [jnp.max(jnp.where(seg == g, h, -jnp.inf), axis=0)
            for g in range(G)]
    gmax = jnp.stack(rows, axis=0)

    @pl.when(i == 0)
    def _():
        gap_ref[...] = gsum
        gsp_ref[...] = gmax
        cnt_ref[...] = cnt

    @pl.when(i > 0)
    def _():
        gap_ref[...] = gap_ref[...] + gsum
        gsp_ref[...] = jnp.maximum(gsp_ref[...], gmax)
        cnt_ref[...] = cnt_ref[...] + cnt


def _bn_rows(x, g, b):
    m = jnp.mean(x, axis=0, keepdims=True)
    v = jnp.mean(x * x, axis=0, keepdims=True) - m * m
    return (x - m) * lax.rsqrt(v + EPS) * g + b


def _mlp_body(gap_ref, gsp_ref, cnt_ref,
              gapg_ref, gapb_ref, gspg_ref, gspb_ref,
              w0_ref, b0_ref, g0_ref, bb0_ref,
              w1_ref, b1_ref, g1_ref, bb1_ref,
              wl_ref, bl_ref, out_ref):
    gap = gap_ref[...] / jnp.maximum(cnt_ref[...], 1.0)
    gap = _bn_rows(gap, gapg_ref[...], gapb_ref[...])
    gsp = _bn_rows(gsp_ref[...], gspg_ref[...], gspb_ref[...])
    out = jnp.concatenate([gap, gsp], axis=1)
    out = jnp.dot(out, w0_ref[...], preferred_element_type=jnp.float32) + b0_ref[...]
    out = (out - jnp.min(out)) / (jnp.max(out) - jnp.min(out))
    out = jnp.maximum(out, 0.0)
    out = _bn_rows(out, g0_ref[...], bb0_ref[...])
    out = jnp.dot(out, w1_ref[...], preferred_element_type=jnp.float32) + b1_ref[...]
    out = (out - jnp.min(out)) / (jnp.max(out) - jnp.min(out))
    out = jnp.maximum(out, 0.0)
    out = _bn_rows(out, g1_ref[...], bb1_ref[...])
    out_ref[...] = (jnp.dot(out, wl_ref[...], preferred_element_type=jnp.float32)
                    + bl_ref[...])


def _row_spec(r, c=D):
    return pl.BlockSpec((r, c), lambda i: (i, 0))


def _full_spec(shape):
    return pl.BlockSpec(shape, lambda i: tuple(0 for _ in shape))


def _kqvs_first(x, wcat, bcat):
    return pl.pallas_call(
        _kqvs_first_body,
        grid=(NBLK,),
        in_specs=[_row_spec(BLK), _full_spec((D, 4 * D)), _full_spec((1, 4 * D))],
        out_specs=[_row_spec(BLK)] * 4,
        out_shape=[jax.ShapeDtypeStruct((N, D), jnp.float32)] * 4,
    )(x, wcat, bcat)


def _kqvs_bn(hp, st, g, b, wcat, bcat):
    return pl.pallas_call(
        _kqvs_bn_body,
        grid=(NBLK,),
        in_specs=[_row_spec(BLK), _full_spec((8, D)), _full_spec((1, D)),
                  _full_spec((1, D)), _full_spec((D, 4 * D)),
                  _full_spec((1, 4 * D))],
        out_specs=[_row_spec(BLK)] * 4,
        out_shape=[jax.ShapeDtypeStruct((N, D), jnp.float32)] * 4,
    )(hp, st, g, b, wcat, bcat)


def _res_stats(agg2, skip):
    return pl.pallas_call(
        _res_stats_body,
        grid=(NBLK,),
        in_specs=[_row_spec(BLK),
                  pl.BlockSpec((BLK, D), lambda i: (i + NBLK, 0)),
                  _row_spec(BLK)],
        out_specs=[_row_spec(BLK), _full_spec((8, D))],
        out_shape=[jax.ShapeDtypeStruct((N, D), jnp.float32),
                   jax.ShapeDtypeStruct((8, D), jnp.float32)],
    )(agg2, agg2, skip)


def _pool(hp, st, g, b, segb):
    return pl.pallas_call(
        _pool_body,
        grid=(NBLK,),
        in_specs=[_row_spec(BLK), _full_spec((8, D)), _full_spec((1, D)),
                  _full_spec((1, D)), _row_spec(BLK)],
        out_specs=[_full_spec((G, D))] * 3,
        out_shape=[jax.ShapeDtypeStruct((G, D), jnp.float32)] * 3,
    )(hp, st, g, b, segb)


def _mlp(gap, gsp, cnt, p):
    w0 = p['lin0_W']
    w1 = p['lin1_W']
    wl = jnp.pad(p['last_W'], ((0, 0), (0, D - 10)))
    bl = jnp.pad(p['last_b'], (0, D - 10)).reshape(1, D)
    args = [gap, gsp, cnt,
            p['gap_g'].reshape(1, D), p['gap_b'].reshape(1, D),
            p['gsp_g'].reshape(1, D), p['gsp_b'].reshape(1, D),
            w0, p['lin0_b'].reshape(1, D),
            p['hbn0_g'].reshape(1, D), p['hbn0_b'].reshape(1, D),
            w1, p['lin1_b'].reshape(1, 64),
            p['hbn1_g'].reshape(1, 64), p['hbn1_b'].reshape(1, 64),
            wl, bl]
    out = pl.pallas_call(
        _mlp_body,
        grid=(1,),
        in_specs=[_full_spec(a.shape) for a in args],
        out_specs=_full_spec((G, D)),
        out_shape=jax.ShapeDtypeStruct((G, D), jnp.float32),
    )(*args)
    return out[:, :10]


def _interleave_perm():
    import numpy as _np
    perm = _np.empty((D,), dtype=_np.int32)
    for g in range(D // 32):
        for i in range(16):
            perm[32 * g + 2 * i] = 32 * g + i
            perm[32 * g + 2 * i + 1] = 32 * g + 16 + i
    return perm


_PERM = _interleave_perm()


def kernel(x, edge_index, batch, params):
    src = edge_index[0]
    dst = edge_index[1]
    segb = jnp.broadcast_to(batch[:, None], (N, D))

    k = q = v = skip = None
    hp = st = None
    for i in range(2):
        wcat = jnp.concatenate(
            [params[f'conv{i}_Wk'], params[f'conv{i}_Wq'][:, _PERM],
             params[f'conv{i}_Wv'][:, _PERM], params[f'conv{i}_Wskip']],
            axis=1)
        bcat = jnp.concatenate(
            [params[f'conv{i}_bk'], params[f'conv{i}_bq'][_PERM],
             params[f'conv{i}_bv'][_PERM], params[f'conv{i}_bias']]
        ).reshape(1, 4 * D)
        if i == 0:
            k, q, v, skip = _kqvs_first(x, wcat, bcat)
        else:
            k, q, v, skip = _kqvs_bn(hp, st, params[f'cbn{i-1}_g'].reshape(1, D),
                                     params[f'cbn{i-1}_b'].reshape(1, D),
                                     wcat, bcat)
        agg2 = _edge_sc(k, q, v, src, dst)
        hp, st = _res_stats(agg2, skip)

    gap, gsp, cnt = _pool(hp, st, params['cbn1_g'].reshape(1, D),
                          params['cbn1_b'].reshape(1, D), segb)
    return _mlp(gap, gsp, cnt, params)


# speedup vs baseline: 4.9460x; 1.2241x over previous
import functools

import jax
import jax.numpy as jnp
from jax import lax
from jax.experimental import pallas as pl
from jax.experimental.pallas import tpu as pltpu
from jax.experimental.pallas import tpu_sc as plsc

N = 10000
D = 128
E = 320000
G = 64
EPS = 1e-5

NW = 32
EPW = E // NW
CHUNK = 40
NCHUNK = EPW // CHUNK
ROWS_PER_TILE = N // 16
BLK = 2000
NBLK = N // BLK


def _edge_body(k_hbm, qv_hbm, src_hbm, dst_hbm, out_hbm,
               acc_sh,
               ixd_a, ixs_a, kb_a, qv_a, m_a, ixc_a,
               ixd_b, ixs_b, kb_b, qv_b, m_b, ixc_b,
               sem_ix_a, sem_g_a, sem_s_a, sem_ix_b, sem_g_b, sem_s_b,
               sem_ixc_a, sem_ixc_b):
    c = lax.axis_index("c")
    s = lax.axis_index("s")
    wid = c * 16 + s
    m_v = m_a

    A = (ixd_a, ixs_a, kb_a, qv_a, sem_ix_a, sem_g_a,
         m_a, ixc_a, sem_s_a, sem_ixc_a)
    B = (ixd_b, ixs_b, kb_b, qv_b, sem_ix_b, sem_g_b,
         m_b, ixc_b, sem_s_b, sem_ixc_b)

    def _zrow(r, carry):
        for j in range(8):
            m_v[r, pl.ds(j * 16, 16)] = jnp.zeros((16,), jnp.float32)
        return carry
    lax.fori_loop(0, CHUNK, _zrow, 0)
    row0 = s * ROWS_PER_TILE
    for t in range(ROWS_PER_TILE // CHUNK):
        pltpu.sync_copy(m_v, acc_sh.at[pl.ds(row0 + t * CHUNK, CHUNK)])
    _rem = ROWS_PER_TILE % CHUNK
    pltpu.sync_copy(m_v.at[pl.ds(0, _rem)],
                    acc_sh.at[pl.ds(row0 + ROWS_PER_TILE - _rem, _rem)])
    plsc.subcore_barrier()

    base_w = wid * EPW

    def _issue_idx(ci, S):
        ixd, ixs, sem_ix = S[0], S[1], S[4]
        eb = base_w + ci * CHUNK
        pltpu.async_copy(dst_hbm.at[pl.ds(eb, CHUNK)], ixd, sem_ix)
        pltpu.async_copy(src_hbm.at[pl.ds(eb, CHUNK)], ixs, sem_ix)

    def _wait_idx(S):
        ixd, ixs, sem_ix = S[0], S[1], S[4]
        pltpu.make_async_copy(dst_hbm.at[pl.ds(base_w, CHUNK)], ixd,
                              sem_ix).wait()
        pltpu.make_async_copy(src_hbm.at[pl.ds(base_w, CHUNK)], ixs,
                              sem_ix).wait()

    def _issue_gathers(S):
        ixd, ixs, kb_v, qv_v, sem_g = S[0], S[1], S[2], S[3], S[5]
        pltpu.async_copy(k_hbm.at[ixd], kb_v, sem_g)
        pltpu.async_copy(qv_hbm.at[ixs], qv_v, sem_g)

    def _wait_gathers(S):
        ixd, ixs, kb_v, qv_v, sem_g = S[0], S[1], S[2], S[3], S[5]
        pltpu.make_async_copy(k_hbm.at[ixd], kb_v, sem_g).wait()
        pltpu.make_async_copy(qv_hbm.at[ixs], qv_v, sem_g).wait()

    def _wait_scatter(S):
        m, ixc, sem_s = S[6], S[7], S[8]
        pltpu.make_async_copy(m, acc_sh.at[ixc], sem_s).wait()

    def _do_chunk(i, S, S_next):
        kb_v, qv_v = S[2], S[3]
        m, ixc, sem_s, sem_ixc = S[6], S[7], S[8], S[9]
        eb = base_w + i * CHUNK

        _wait_gathers(S)

        @pl.when(i >= 2)
        def _():
            _wait_scatter(S)

        pltpu.async_copy(dst_hbm.at[pl.ds(eb, CHUNK)], ixc, sem_ixc)

        @pl.when(i + 2 < NCHUNK)
        def _():
            _issue_idx(i + 2, S)

        @pl.when(i + 1 < NCHUNK)
        def _():
            _wait_idx(S_next)
            _issue_gathers(S_next)

        hi_mask = jnp.full((16,), -65536, jnp.int32)

        def _bf2f(w):
            lo = lax.bitcast_convert_type(jnp.left_shift(w, 16), jnp.float32)
            hi = lax.bitcast_convert_type(jnp.bitwise_and(w, hi_mask),
                                          jnp.float32)
            return lo, hi

        @plsc.parallel_loop(0, CHUNK, 1, unroll=4)
        def _rowp(e):
            for g in range(4):
                q0, q1 = _bf2f(qv_v[e, pl.ds(g * 16, 16)])
                v0, v1 = _bf2f(qv_v[e, pl.ds(64 + g * 16, 16)])
                s0 = kb_v[e, pl.ds(g * 16, 16)] + q0
                s1 = kb_v[e, pl.ds(64 + g * 16, 16)] + q1
                m[e, pl.ds(g * 16, 16)] = v0 / (1.0 + jnp.exp(-s0))
                m[e, pl.ds(64 + g * 16, 16)] = v1 / (1.0 + jnp.exp(-s1))

        pltpu.make_async_copy(dst_hbm.at[pl.ds(eb, CHUNK)], ixc,
                              sem_ixc).wait()
        pltpu.async_copy(m, acc_sh.at[ixc], sem_s, add=True)

    pltpu.sync_copy(dst_hbm.at[pl.ds(base_w, CHUNK)], ixd_a)
    pltpu.sync_copy(src_hbm.at[pl.ds(base_w, CHUNK)], ixs_a)
    _issue_gathers(A)
    _issue_idx(1, B)

    def _pair(t, carry):
        _do_chunk(2 * t, A, B)
        _do_chunk(2 * t + 1, B, A)
        return carry

    lax.fori_loop(0, NCHUNK // 2, _pair, 0)
    _wait_scatter(A)
    _wait_scatter(B)
    plsc.subcore_barrier()

    pltpu.sync_copy(acc_sh.at[pl.ds(s * ROWS_PER_TILE, ROWS_PER_TILE)],
                    out_hbm.at[c, s])


@functools.cache
def _make_edge_sc():
  return pl.kernel(
    _edge_body,
    out_type=jax.ShapeDtypeStruct((2, 16, ROWS_PER_TILE, D), jnp.float32),
    mesh=plsc.VectorSubcoreMesh(core_axis_name="c", subcore_axis_name="s",
                                num_cores=2, num_subcores=16),
    scratch_types=[
        pltpu.VMEM_SHARED((N, D), jnp.float32),
        pltpu.VMEM((CHUNK,), jnp.int32),
        pltpu.VMEM((CHUNK,), jnp.int32),
        pltpu.VMEM((CHUNK, D), jnp.float32),
        pltpu.VMEM((CHUNK, D), jnp.int32),
        pltpu.VMEM((CHUNK, D), jnp.float32),
        pltpu.VMEM((CHUNK,), jnp.int32),
        pltpu.VMEM((CHUNK,), jnp.int32),
        pltpu.VMEM((CHUNK,), jnp.int32),
        pltpu.VMEM((CHUNK, D), jnp.float32),
        pltpu.VMEM((CHUNK, D), jnp.int32),
        pltpu.VMEM((CHUNK, D), jnp.float32),
        pltpu.VMEM((CHUNK,), jnp.int32),
        pltpu.SemaphoreType.DMA,
        pltpu.SemaphoreType.DMA,
        pltpu.SemaphoreType.DMA,
        pltpu.SemaphoreType.DMA,
        pltpu.SemaphoreType.DMA,
        pltpu.SemaphoreType.DMA,
        pltpu.SemaphoreType.DMA,
        pltpu.SemaphoreType.DMA,
    ],
  )


def _edge_sc(k, qv, src, dst):
    out = _make_edge_sc()(k, qv, src, dst)
    return out.reshape(2 * N, D)


def _bf16_bits(x):
    xb = lax.bitcast_convert_type(x, jnp.int32)
    rnd = xb + 0x7FFF + jnp.bitwise_and(lax.shift_right_logical(xb, 16), 1)
    return jnp.bitwise_and(lax.shift_right_logical(rnd, 16), 0xFFFF)


def _pack_qv(q, v):
    def _pk(t):
        lo = _bf16_bits(t[:, 0:D // 2])
        hi = _bf16_bits(t[:, D // 2:D])
        return jnp.bitwise_or(lo, lax.shift_left(hi, 16))
    return jnp.concatenate([_pk(q), _pk(v)], axis=1)


def _kqvs_first_body(x_ref, w_ref, b_ref, k_ref, qv_ref, sk_ref):
    y = jnp.dot(x_ref[...], w_ref[...], preferred_element_type=jnp.float32)
    y = y + b_ref[...]
    k_ref[...] = y[:, 0:D]
    qv_ref[...] = _pack_qv(y[:, D:2 * D], y[:, 2 * D:3 * D])
    sk_ref[...] = y[:, 3 * D:4 * D]


def _kqvs_bn_body(x_ref, st_ref, g_ref, bb_ref, w_ref, b_ref,
                  k_ref, qv_ref, sk_ref):
    mean = st_ref[0:1, :] * (1.0 / N)
    var = st_ref[1:2, :] * (1.0 / N) - mean * mean
    h = (x_ref[...] - mean) * lax.rsqrt(var + EPS) * g_ref[...] + bb_ref[...]
    y = jnp.dot(h, w_ref[...], preferred_element_type=jnp.float32)
    y = y + b_ref[...]
    k_ref[...] = y[:, 0:D]
    qv_ref[...] = _pack_qv(y[:, D:2 * D], y[:, 2 * D:3 * D])
    sk_ref[...] = y[:, 3 * D:4 * D]


def _res_stats_body(a0_ref, a1_ref, sk_ref, hp_ref, st_ref):
    i = pl.program_id(0)
    hp = a0_ref[...] + a1_ref[...] + sk_ref[...]
    hp_ref[...] = hp
    s1 = jnp.sum(hp, axis=0, keepdims=True)
    s2 = jnp.sum(hp * hp, axis=0, keepdims=True)
    blk = jnp.concatenate([s1, s2, jnp.zeros((6, D), jnp.float32)], axis=0)

    @pl.when(i == 0)
    def _():
        st_ref[...] = blk

    @pl.when(i > 0)
    def _():
        st_ref[...] = st_ref[...] + blk


def _pool_body(hp_ref, st_ref, g_ref, bb_ref, seg_ref,
               gap_ref, gsp_ref, cnt_ref):
    i = pl.program_id(0)
    mean = st_ref[0:1, :] * (1.0 / N)
    var = st_ref[1:2, :] * (1.0 / N) - mean * mean
    h = (hp_ref[...] - mean) * lax.rsqrt(var + EPS) * g_ref[...] + bb_ref[...]
    seg = seg_ref[:, 0:1]
    segT = seg.reshape(1, BLK)
    gid = lax.broadcasted_iota(jnp.int32, (G, BLK), 0)
    mf = (gid == segT).astype(jnp.float32)
    gsum = jnp.dot(mf, h, preferred_element_type=jnp.float32)
    cnt = jnp.broadcast_to(jnp.sum(mf, axis=1, keepdims=True), (G, D))
    rows = [jnp.max(jnp.where(seg == g, h, -jnp.inf), axis=0)
            for g in range(G)]
    gmax = jnp.stack(rows, axis=0)

    @pl.when(i == 0)
    def _():
        gap_ref[...] = gsum
        gsp_ref[...] = gmax
        cnt_ref[...] = cnt

    @pl.when(i > 0)
    def _():
        gap_ref[...] = gap_ref[...] + gsum
        gsp_ref[...] = jnp.maximum(gsp_ref[...], gmax)
        cnt_ref[...] = cnt_ref[...] + cnt


def _bn_rows(x, g, b):
    m = jnp.mean(x, axis=0, keepdims=True)
    v = jnp.mean(x * x, axis=0, keepdims=True) - m * m
    return (x - m) * lax.rsqrt(v + EPS) * g + b


def _mlp_body(gap_ref, gsp_ref, cnt_ref,
              gapg_ref, gapb_ref, gspg_ref, gspb_ref,
              w0_ref, b0_ref, g0_ref, bb0_ref,
              w1_ref, b1_ref, g1_ref, bb1_ref,
              wl_ref, bl_ref, out_ref):
    gap = gap_ref[...] / jnp.maximum(cnt_ref[...], 1.0)
    gap = _bn_rows(gap, gapg_ref[...], gapb_ref[...])
    gsp = _bn_rows(gsp_ref[...], gspg_ref[...], gspb_ref[...])
    out = jnp.concatenate([gap, gsp], axis=1)
    out = jnp.dot(out, w0_ref[...], preferred_element_type=jnp.float32) + b0_ref[...]
    out = (out - jnp.min(out)) / (jnp.max(out) - jnp.min(out))
    out = jnp.maximum(out, 0.0)
    out = _bn_rows(out, g0_ref[...], bb0_ref[...])
    out = jnp.dot(out, w1_ref[...], preferred_element_type=jnp.float32) + b1_ref[...]
    out = (out - jnp.min(out)) / (jnp.max(out) - jnp.min(out))
    out = jnp.maximum(out, 0.0)
    out = _bn_rows(out, g1_ref[...], bb1_ref[...])
    out_ref[...] = (jnp.dot(out, wl_ref[...], preferred_element_type=jnp.float32)
                    + bl_ref[...])


def _row_spec(r, c=D):
    return pl.BlockSpec((r, c), lambda i: (i, 0))


def _full_spec(shape):
    return pl.BlockSpec(shape, lambda i: tuple(0 for _ in shape))


_KQVS_OUT = [jax.ShapeDtypeStruct((N, D), jnp.float32),
             jax.ShapeDtypeStruct((N, D), jnp.int32),
             jax.ShapeDtypeStruct((N, D), jnp.float32)]


def _kqvs_first(x, wcat, bcat):
    return pl.pallas_call(
        _kqvs_first_body,
        grid=(NBLK,),
        in_specs=[_row_spec(BLK), _full_spec((D, 4 * D)), _full_spec((1, 4 * D))],
        out_specs=[_row_spec(BLK)] * 3,
        out_shape=_KQVS_OUT,
    )(x, wcat, bcat)


def _kqvs_bn(hp, st, g, b, wcat, bcat):
    return pl.pallas_call(
        _kqvs_bn_body,
        grid=(NBLK,),
        in_specs=[_row_spec(BLK), _full_spec((8, D)), _full_spec((1, D)),
                  _full_spec((1, D)), _full_spec((D, 4 * D)),
                  _full_spec((1, 4 * D))],
        out_specs=[_row_spec(BLK)] * 3,
        out_shape=_KQVS_OUT,
    )(hp, st, g, b, wcat, bcat)


def _res_stats(agg2, skip):
    return pl.pallas_call(
        _res_stats_body,
        grid=(NBLK,),
        in_specs=[_row_spec(BLK),
                  pl.BlockSpec((BLK, D), lambda i: (i + NBLK, 0)),
                  _row_spec(BLK)],
        out_specs=[_row_spec(BLK), _full_spec((8, D))],
        out_shape=[jax.ShapeDtypeStruct((N, D), jnp.float32),
                   jax.ShapeDtypeStruct((8, D), jnp.float32)],
    )(agg2, agg2, skip)


def _pool(hp, st, g, b, segb):
    return pl.pallas_call(
        _pool_body,
        grid=(NBLK,),
        in_specs=[_row_spec(BLK), _full_spec((8, D)), _full_spec((1, D)),
                  _full_spec((1, D)), _row_spec(BLK)],
        out_specs=[_full_spec((G, D))] * 3,
        out_shape=[jax.ShapeDtypeStruct((G, D), jnp.float32)] * 3,
    )(hp, st, g, b, segb)


def _mlp(gap, gsp, cnt, p):
    w0 = p['lin0_W']
    w1 = p['lin1_W']
    wl = jnp.pad(p['last_W'], ((0, 0), (0, D - 10)))
    bl = jnp.pad(p['last_b'], (0, D - 10)).reshape(1, D)
    args = [gap, gsp, cnt,
            p['gap_g'].reshape(1, D), p['gap_b'].reshape(1, D),
            p['gsp_g'].reshape(1, D), p['gsp_b'].reshape(1, D),
            w0, p['lin0_b'].reshape(1, D),
            p['hbn0_g'].reshape(1, D), p['hbn0_b'].reshape(1, D),
            w1, p['lin1_b'].reshape(1, 64),
            p['hbn1_g'].reshape(1, 64), p['hbn1_b'].reshape(1, 64),
            wl, bl]
    out = pl.pallas_call(
        _mlp_body,
        grid=(1,),
        in_specs=[_full_spec(a.shape) for a in args],
        out_specs=_full_spec((G, D)),
        out_shape=jax.ShapeDtypeStruct((G, D), jnp.float32),
    )(*args)
    return out[:, :10]


def kernel(x, edge_index, batch, params):
    src = edge_index[0]
    dst = edge_index[1]
    segb = jnp.broadcast_to(batch[:, None], (N, D))

    k = q = v = skip = None
    hp = st = None
    for i in range(2):
        wcat = jnp.concatenate(
            [params[f'conv{i}_Wk'], params[f'conv{i}_Wq'],
             params[f'conv{i}_Wv'], params[f'conv{i}_Wskip']],
            axis=1)
        bcat = jnp.concatenate(
            [params[f'conv{i}_bk'], params[f'conv{i}_bq'],
             params[f'conv{i}_bv'], params[f'conv{i}_bias']]
        ).reshape(1, 4 * D)
        if i == 0:
            k, qv, skip = _kqvs_first(x, wcat, bcat)
        else:
            k, qv, skip = _kqvs_bn(hp, st, params[f'cbn{i-1}_g'].reshape(1, D),
                                   params[f'cbn{i-1}_b'].reshape(1, D),
                                   wcat, bcat)
        agg2 = _edge_sc(k, qv, src, dst)
        hp, st = _res_stats(agg2, skip)

    gap, gsp, cnt = _pool(hp, st, params['cbn1_g'].reshape(1, D),
                          params['cbn1_b'].reshape(1, D), segb)
    return _mlp(gap, gsp, cnt, params)
